# baseline (speedup 1.0000x reference)
import functools

import jax
import jax.numpy as jnp
from jax import lax
from jax.experimental import pallas as pl
from jax.experimental.pallas import tpu as pltpu
from jax.experimental.pallas import tpu_sc as plsc

EMBED_DIM = 64
PAIR = 2 * EMBED_DIM
NUM_CORES = 2
NUM_SUBCORES = 16
NUM_WORKERS = NUM_CORES * NUM_SUBCORES
PASS_ROWS = 64
LANES = 16


def _newton_sqrt(x):
    i = plsc.bitcast(x, jnp.int32)
    y = plsc.bitcast(jnp.int32(0x5F3759DF) - (i >> 1), jnp.float32)
    y = y * (1.5 - 0.5 * x * y * y)
    y = y * (1.5 - 0.5 * x * y * y)
    y = y * (1.5 - 0.5 * x * y * y)
    return jnp.where(x > 0.0, x * y, 0.0)


def _make_sc_kernel(batch, n_rel_pairs):
    bpw = batch // NUM_WORKERS
    n_passes = bpw // PASS_ROWS
    groups = PASS_ROWS // LANES
    mesh = plsc.VectorSubcoreMesh(core_axis_name="c", subcore_axis_name="s")

    @functools.partial(
        pl.kernel,
        mesh=mesh,
        compiler_params=pltpu.CompilerParams(needs_layout_passes=False),
        out_type=jax.ShapeDtypeStruct((batch,), jnp.float32),
        scratch_types=[
            pltpu.VMEM((n_passes, PASS_ROWS), jnp.int32),
            pltpu.VMEM((n_passes, PASS_ROWS), jnp.int32),
            pltpu.VMEM((n_passes, PASS_ROWS), jnp.int32),
            pltpu.VMEM((n_passes, PASS_ROWS), jnp.int32),
            pltpu.VMEM((n_passes, PASS_ROWS), jnp.int32),
            pltpu.VMEM((n_passes, PASS_ROWS), jnp.int32),
            pltpu.VMEM((n_rel_pairs, PAIR), jnp.float32),
            pltpu.VMEM((PASS_ROWS, EMBED_DIM), jnp.float32),
            pltpu.VMEM((PASS_ROWS, EMBED_DIM), jnp.float32),
            pltpu.VMEM((PASS_ROWS, EMBED_DIM), jnp.float32),
            pltpu.VMEM((PASS_ROWS, EMBED_DIM), jnp.float32),
            pltpu.VMEM((bpw,), jnp.float32),
            pltpu.SemaphoreType.DMA,
            pltpu.SemaphoreType.DMA,
            pltpu.SemaphoreType.DMA,
        ],
    )
    def sc_kernel(ent_hbm, rel_hbm, hb_hbm, hs_hbm, tb_hbm, ts_hbm,
                  rp_hbm, ro_hbm, dummy_hbm, out_hbm,
                  hblk, hsub, tblk, tsub, rpai, roff, relv,
                  hb0, hb1, tb0, tb1, outv, sem0, sem1, semr):
        wid = lax.axis_index("s") * NUM_CORES + lax.axis_index("c")
        sl_w = pl.ds(wid * n_passes, n_passes)
        rel_cp = pltpu.async_copy(rel_hbm, relv, semr)
        pltpu.sync_copy(hb_hbm.at[sl_w], hblk)
        pltpu.sync_copy(hs_hbm.at[sl_w], hsub)
        pltpu.sync_copy(tb_hbm.at[sl_w], tblk)
        pltpu.sync_copy(ts_hbm.at[sl_w], tsub)
        pltpu.sync_copy(rp_hbm.at[sl_w], rpai)
        pltpu.sync_copy(ro_hbm.at[sl_w], roff)

        hbufs, tbufs = (hb0, hb1), (tb0, tb1)
        sems = (sem0, sem1)
        lane = lax.iota(jnp.int32, LANES)

        def fire(p):
            s = p % 2
            hbuf, tbuf, sem = hbufs[s], tbufs[s], sems[s]

            def body(k, carry):
                hbv = hblk[p, pl.ds(k * LANES, LANES)]
                hsv = hsub[p, pl.ds(k * LANES, LANES)]
                tbv = tblk[p, pl.ds(k * LANES, LANES)]
                tsv = tsub[p, pl.ds(k * LANES, LANES)]
                for u in range(LANES):
                    lrow = k * LANES + u
                    pltpu.async_copy(ent_hbm.at[hbv[u], hsv[u]],
                                     hbuf.at[lrow], sem)
                    pltpu.async_copy(ent_hbm.at[tbv[u], tsv[u]],
                                     tbuf.at[lrow], sem)
                return carry

            lax.fori_loop(0, groups, body, 0)

        def drain(p):
            s = p % 2
            for buf in (hbufs[s], tbufs[s]):
                pltpu.make_async_copy(dummy_hbm, buf, sems[s]).wait()

        def compute(p):
            s = p % 2
            hbuf, tbuf = hbufs[s], tbufs[s]

            def group_body(g, carry):
                row0 = g * LANES
                rpv = rpai[p, pl.ds(row0, LANES)]
                rov = roff[p, pl.ds(row0, LANES)]
                vec = jnp.zeros((LANES,), jnp.float32)
                for u in range(LANES):
                    b = row0 + u
                    rp_, ro_ = rpv[u], rov[u]
                    acc = jnp.zeros((LANES,), jnp.float32)
                    for c in range(EMBED_DIM // LANES):
                        sl = pl.ds(c * LANES, LANES)
                        diff = (hbuf[b, sl]
                                + relv[rp_, pl.ds(ro_ + c * LANES, LANES)]
                                - tbuf[b, sl])
                        acc = acc + diff * diff
                    vec = jnp.where(lane == u, jnp.sum(acc), vec)
                outv[pl.ds(p * PASS_ROWS + row0, LANES)] = -_newton_sqrt(vec)
                return carry

            lax.fori_loop(0, groups, group_body, 0)

        fire(0)
        rel_cp.wait()
        for p in range(n_passes):
            if p + 1 < n_passes:
                fire(p + 1)
            drain(p)
            compute(p)

        pltpu.sync_copy(outv, out_hbm.at[pl.ds(wid * bpw, bpw)])

    return sc_kernel


def kernel(entity_table, relation_table, h, r, t):
    batch = h.shape[0]
    ent3 = entity_table.reshape(entity_table.shape[0] // 8, 8, EMBED_DIM)
    n_rel_pairs = relation_table.shape[0] // 2
    rel2 = relation_table.reshape(n_rel_pairs, PAIR)
    shape2 = (NUM_WORKERS * (batch // NUM_WORKERS // PASS_ROWS), PASS_ROWS)

    h32 = h.astype(jnp.int32)
    t32 = t.astype(jnp.int32)
    r32 = r.astype(jnp.int32)
    hb_ = (h32 >> 3).reshape(shape2)
    hs_ = (h32 & 7).reshape(shape2)
    tb_ = (t32 >> 3).reshape(shape2)
    ts_ = (t32 & 7).reshape(shape2)
    rp_ = (r32 >> 1).reshape(shape2)
    ro_ = ((r32 & 1) * EMBED_DIM).reshape(shape2)
    dummy = jnp.zeros((PASS_ROWS, EMBED_DIM), jnp.float32)
    return _make_sc_kernel(batch, n_rel_pairs)(
        ent3, rel2, hb_, hs_, tb_, ts_, rp_, ro_, dummy)
